# baseline (speedup 1.0000x reference)
import functools

import jax
import jax.numpy as jnp
from jax import lax
from jax.experimental import pallas as pl
from jax.experimental.pallas import tpu as pltpu
from jax.experimental.pallas import tpu_sc as plsc

NSEG = 10000
D = 128
N = 320000
NC = 2
NS = 16
NW = NC * NS
ROWS_PER_TILE = N // NW
CHUNK = 125
NCHUNK = ROWS_PER_TILE // CHUNK
SEG_PER_TILE = NSEG // NS
CW = 16
W = D + CW

_mesh = plsc.VectorSubcoreMesh(core_axis_name="c", subcore_axis_name="s")


@functools.partial(
    pl.kernel,
    mesh=_mesh,
    compiler_params=pltpu.CompilerParams(use_tc_tiling_on_sc=False),
    out_type=jax.ShapeDtypeStruct((NC * NSEG, W), jnp.float32),
    scratch_types=[
        pltpu.VMEM((1, CHUNK), jnp.int32),
        pltpu.VMEM((1, CHUNK), jnp.int32),
        pltpu.VMEM((CHUNK, W), jnp.float32),
        pltpu.VMEM((CHUNK, W), jnp.float32),
        pltpu.VMEM_SHARED((NSEG, W), jnp.float32),
        pltpu.SemaphoreType.DMA,
        pltpu.SemaphoreType.DMA,
        pltpu.SemaphoreType.DMA,
        pltpu.SemaphoreType.DMA,
    ],
)
def _scatter_stage(data_hbm, ids_hbm, ztpl_hbm, otpl_hbm, pacc_hbm,
                   ids_a, ids_b, rows_a, rows_b, sacc,
                   sem_a, sem_b, sem_ia, sem_ib):
    cid = lax.axis_index("c")
    sid = lax.axis_index("s")
    wid = sid * NC + cid
    row0 = wid * ROWS_PER_TILE
    id0 = wid * NCHUNK
    seg0 = sid * SEG_PER_TILE

    pltpu.sync_copy(ztpl_hbm, rows_a)
    for k in range(SEG_PER_TILE // CHUNK):
        pltpu.sync_copy(rows_a, sacc.at[pl.ds(seg0 + k * CHUNK, CHUNK)])
    pltpu.sync_copy(otpl_hbm, rows_a)
    pltpu.sync_copy(otpl_hbm, rows_b)
    plsc.subcore_barrier()

    pltpu.async_copy(data_hbm.at[pl.ds(row0, CHUNK)],
                     rows_a.at[:, pl.ds(0, D)], sem_a)
    pltpu.async_copy(ids_hbm.at[pl.ds(id0, 1)], ids_a, sem_ia)
    pltpu.async_copy(data_hbm.at[pl.ds(row0 + CHUNK, CHUNK)],
                     rows_b.at[:, pl.ds(0, D)], sem_b)
    pltpu.async_copy(ids_hbm.at[pl.ds(id0 + 1, 1)], ids_b, sem_ib)

    def step(i, carry):
        bufs = ((rows_a, ids_a, sem_a, sem_ia), (rows_b, ids_b, sem_b, sem_ib))
        for b, (buf, idb, sem, isem) in enumerate(bufs):
            j = i * 2 + b
            pltpu.make_async_copy(data_hbm.at[pl.ds(row0, CHUNK)],
                                  buf.at[:, pl.ds(0, D)], sem).wait()
            pltpu.make_async_copy(ids_hbm.at[pl.ds(id0, 1)], idb, isem).wait()
            pltpu.sync_copy(buf, sacc.at[idb.at[0]], add=True)

            @pl.when(j + 2 < NCHUNK)
            def _():
                pltpu.async_copy(
                    data_hbm.at[pl.ds(row0 + (j + 2) * CHUNK, CHUNK)],
                    buf.at[:, pl.ds(0, D)], sem)
                pltpu.async_copy(ids_hbm.at[pl.ds(id0 + j + 2, 1)], idb, isem)
        return carry

    lax.fori_loop(0, NCHUNK // 2, step, 0)
    plsc.subcore_barrier()

    out0 = cid * NSEG + seg0
    for k in range(SEG_PER_TILE // CHUNK):
        pltpu.sync_copy(sacc.at[pl.ds(seg0 + k * CHUNK, CHUNK)], rows_a)
        pltpu.sync_copy(rows_a, pacc_hbm.at[pl.ds(out0 + k * CHUNK, CHUNK)])


_FR = 1000


def _fin_body(a_ref, o_ref):
    a = a_ref[0] + a_ref[1]
    s = a[:, :D]
    cnt = jnp.sum(a[:, D:], axis=1, keepdims=True)
    o_ref[...] = s / jnp.maximum(cnt, 1.0)


_finalize = pl.pallas_call(
    _fin_body,
    grid=(NSEG // _FR,),
    in_specs=[pl.BlockSpec((NC, _FR, W), lambda g: (0, g, 0))],
    out_specs=pl.BlockSpec((_FR, D), lambda g: (g, 0)),
    out_shape=jax.ShapeDtypeStruct((NSEG, D), jnp.float32),
)


@jax.jit
def kernel(data, segment_ids):
    ids = segment_ids.astype(jnp.int32).reshape(NW * NCHUNK, CHUNK)
    ztpl = jnp.zeros((CHUNK, W), jnp.float32)
    otpl = jnp.concatenate(
        [jnp.zeros((CHUNK, D), jnp.float32),
         jnp.full((CHUNK, CW), 1.0 / CW, jnp.float32)], axis=1)
    pacc = _scatter_stage(data, ids, ztpl, otpl)
    return _finalize(pacc.reshape(NC, NSEG, W))


# speedup vs baseline: 1.2904x; 1.2904x over previous
import functools

import jax
import jax.numpy as jnp
from jax import lax
from jax.experimental import pallas as pl
from jax.experimental.pallas import tpu as pltpu
from jax.experimental.pallas import tpu_sc as plsc

NSEG = 10000
D = 128
N = 320000
NC = 2
NS = 16
NW = NC * NS
ROWS_PER_TILE = N // NW
CHUNK = 100
NCHUNK = ROWS_PER_TILE // CHUNK
NBUF = 3
SEG_PER_TILE = NSEG // NS
CW = 16
EXCHUNKS = [(0, 100), (100, 100), (200, 100), (300, 100),
            (400, 100), (500, 100), (600, 25)]

_mesh = plsc.VectorSubcoreMesh(core_axis_name="c", subcore_axis_name="s")


@functools.partial(
    pl.kernel,
    mesh=_mesh,
    compiler_params=pltpu.CompilerParams(use_tc_tiling_on_sc=False),
    out_type=[
        jax.ShapeDtypeStruct((NC * NSEG, D), jnp.float32),
        jax.ShapeDtypeStruct((NC * NSEG, CW), jnp.float32),
    ],
    scratch_types=[
        [pltpu.VMEM((1, CHUNK), jnp.int32) for _ in range(NBUF)],
        [pltpu.VMEM((CHUNK, D), jnp.float32) for _ in range(NBUF)],
        pltpu.VMEM((CHUNK, CW), jnp.float32),
        pltpu.VMEM_SHARED((NSEG, D), jnp.float32),
        pltpu.VMEM_SHARED((NSEG, CW), jnp.float32),
        [pltpu.SemaphoreType.DMA for _ in range(NBUF)],
        [pltpu.SemaphoreType.DMA for _ in range(NBUF)],
    ],
)
def _scatter_stage(data_hbm, ids_hbm, zrows_hbm, ones_hbm, zcnt_hbm,
                   psums_hbm, pcnts_hbm,
                   ids_v, rows_v, ones_v, ssum, scnt, sems, isems):
    cid = lax.axis_index("c")
    sid = lax.axis_index("s")
    wid = sid * NC + cid
    row0 = wid * ROWS_PER_TILE
    id0 = wid * NCHUNK
    seg0 = sid * SEG_PER_TILE

    pltpu.sync_copy(zrows_hbm, rows_v[0])
    for off, sz in EXCHUNKS:
        pltpu.sync_copy(rows_v[0].at[pl.ds(0, sz)],
                        ssum.at[pl.ds(seg0 + off, sz)])
    pltpu.sync_copy(zcnt_hbm, ones_v)
    for off, sz in EXCHUNKS:
        pltpu.sync_copy(ones_v.at[pl.ds(0, sz)],
                        scnt.at[pl.ds(seg0 + off, sz)])
    pltpu.sync_copy(ones_hbm, ones_v)
    plsc.subcore_barrier()

    def fire(j, b):
        pltpu.async_copy(data_hbm.at[pl.ds(row0 + j * CHUNK, CHUNK)],
                         rows_v[b], sems[b])
        pltpu.async_copy(ids_hbm.at[pl.ds(id0 + j, 1)], ids_v[b], isems[b])

    def consume(j, b):
        pltpu.make_async_copy(data_hbm.at[pl.ds(row0, CHUNK)],
                              rows_v[b], sems[b]).wait()
        pltpu.make_async_copy(ids_hbm.at[pl.ds(id0, 1)],
                              ids_v[b], isems[b]).wait()
        pltpu.sync_copy(rows_v[b], ssum.at[ids_v[b].at[0]], add=True)
        pltpu.sync_copy(ones_v, scnt.at[ids_v[b].at[0]], add=True)

    for j in range(NBUF):
        fire(j, j)
    consume(0, 0)
    fire(NBUF, 0)

    def step(i, carry):
        for b in range(NBUF):
            j = NBUF * i + 1 + b
            bb = (1 + b) % NBUF
            consume(j, bb)

            @pl.when(j + NBUF < NCHUNK)
            def _():
                fire_j = j + NBUF
                pltpu.async_copy(
                    data_hbm.at[pl.ds(row0 + fire_j * CHUNK, CHUNK)],
                    rows_v[bb], sems[bb])
                pltpu.async_copy(ids_hbm.at[pl.ds(id0 + fire_j, 1)],
                                 ids_v[bb], isems[bb])
        return carry

    lax.fori_loop(0, (NCHUNK - 1) // NBUF, step, 0)
    plsc.subcore_barrier()

    out0 = cid * NSEG + seg0
    for off, sz in EXCHUNKS:
        pltpu.sync_copy(ssum.at[pl.ds(seg0 + off, sz)],
                        rows_v[0].at[pl.ds(0, sz)])
        pltpu.sync_copy(rows_v[0].at[pl.ds(0, sz)],
                        psums_hbm.at[pl.ds(out0 + off, sz)])
    for off, sz in EXCHUNKS:
        pltpu.sync_copy(scnt.at[pl.ds(seg0 + off, sz)],
                        ones_v.at[pl.ds(0, sz)])
        pltpu.sync_copy(ones_v.at[pl.ds(0, sz)],
                        pcnts_hbm.at[pl.ds(out0 + off, sz)])


_FR = 1000


def _fin_body(s_ref, c_ref, o_ref):
    s = s_ref[0] + s_ref[1]
    c = c_ref[0] + c_ref[1]
    cnt = jnp.sum(c, axis=1, keepdims=True)
    o_ref[...] = s / jnp.maximum(cnt, 1.0)


_finalize = pl.pallas_call(
    _fin_body,
    grid=(NSEG // _FR,),
    in_specs=[
        pl.BlockSpec((NC, _FR, D), lambda g: (0, g, 0)),
        pl.BlockSpec((NC, _FR, CW), lambda g: (0, g, 0)),
    ],
    out_specs=pl.BlockSpec((_FR, D), lambda g: (g, 0)),
    out_shape=jax.ShapeDtypeStruct((NSEG, D), jnp.float32),
)


@jax.jit
def kernel(data, segment_ids):
    ids = segment_ids.astype(jnp.int32).reshape(NW * NCHUNK, CHUNK)
    zrows = jnp.zeros((CHUNK, D), jnp.float32)
    ones = jnp.full((CHUNK, CW), 1.0 / CW, jnp.float32)
    zcnt = jnp.zeros((CHUNK, CW), jnp.float32)
    psums, pcnts = _scatter_stage(data, ids, zrows, ones, zcnt)
    return _finalize(psums.reshape(NC, NSEG, D), pcnts.reshape(NC, NSEG, CW))
